# Initial kernel scaffold; baseline (speedup 1.0000x reference)
import jax
import jax.numpy as jnp
from jax.experimental import pallas as pl


def kernel(x, edge_index, W1, b1, W2, b2):
    raise NotImplementedError("write your pallas kernel here")


# speedup vs baseline: 25.5591x; 25.5591x over previous
import functools

import jax
import jax.numpy as jnp
from jax import lax
from jax.experimental import pallas as pl
from jax.experimental.pallas import tpu as pltpu
from jax.experimental.pallas import tpu_sc as plsc

_N = 10000
_E = 320000
_CHUNK = 128
_NC, _NS = 2, 16
_CPT = 80
_SEG = 40
_TOT_CHUNKS = _NC * _NS * _CPT
_EPAD = _TOT_CHUNKS * _CHUNK
_NACC = 10240
_RPT = _NACC // _NS
_ZR = 16
_NB = 2
_RB = 1000


def _vec_mesh():
    return plsc.VectorSubcoreMesh(
        core_axis_name="c", subcore_axis_name="s",
        num_cores=_NC, num_subcores=_NS)


def _fill_rows(ref, rows, width, value):
    vec = jnp.full((16,), value, jnp.float32)

    @pl.loop(0, rows)
    def _(r):
        row = ref.at[r]
        for c in range(width // 16):
            row[pl.ds(c * 16, 16)] = vec


def _make_sc_deg():
    @functools.partial(
        pl.kernel,
        out_type=jax.ShapeDtypeStruct((_NC, _NACC, 16), jnp.float32),
        mesh=_vec_mesh(),
        scratch_types=[
            pltpu.VMEM((_CPT, _CHUNK), jnp.int32),
            pltpu.VMEM((_CHUNK, 16), jnp.float32),
            pltpu.VMEM((_ZR, 16), jnp.float32),
            pltpu.VMEM_SHARED((_NACC, 16), jnp.float32),
            pltpu.SemaphoreType.DMA,
            pltpu.SemaphoreType.DMA,
        ])
    def deg_kernel(dst_hbm, out_hbm, dst_v, ones_v, z_v, acc, sem, sem2):
        cid = lax.axis_index("c")
        sid = lax.axis_index("s")
        w = cid * _NS + sid
        _fill_rows(ones_v, _CHUNK, 16, 1.0)
        _fill_rows(z_v, _ZR, 16, 0.0)

        @pl.loop(0, _RPT // _ZR)
        def _(z):
            pltpu.sync_copy(z_v, acc.at[pl.ds(sid * _RPT + z * _ZR, _ZR)])

        pltpu.sync_copy(dst_hbm.at[pl.ds(w * _CPT, _CPT)], dst_v)
        plsc.subcore_barrier()

        @pl.loop(0, _CPT, step=2)
        def _(j):
            pltpu.async_copy(ones_v, acc.at[dst_v.at[j]], sem, add=True)
            pltpu.async_copy(ones_v, acc.at[dst_v.at[j + 1]], sem2, add=True)
            pltpu.make_async_copy(ones_v, acc.at[dst_v.at[j]], sem).wait()
            pltpu.make_async_copy(ones_v, acc.at[dst_v.at[j + 1]], sem2).wait()

        plsc.subcore_barrier()
        pltpu.sync_copy(acc.at[pl.ds(sid * _RPT, _RPT)],
                        out_hbm.at[cid, pl.ds(sid * _RPT, _RPT)])

    return deg_kernel


def _make_sc_agg(d):
    @functools.partial(
        pl.kernel,
        out_type=jax.ShapeDtypeStruct((_NC, _NACC, d), jnp.float32),
        mesh=_vec_mesh(),
        scratch_types=[
            pltpu.VMEM((_SEG, _CHUNK), jnp.int32),
            pltpu.VMEM((_SEG, _CHUNK), jnp.int32),
            pltpu.VMEM((_NB, _CHUNK, d), jnp.float32),
            pltpu.VMEM_SHARED((_NACC, d), jnp.float32),
            pltpu.SemaphoreType.DMA((_NB,)),
            pltpu.SemaphoreType.DMA((_NB,)),
        ])
    def agg_kernel(g_hbm, src_hbm, dst_hbm, out_hbm,
                   src_v, dst_v, gbuf, acc, gsem, ssem):
        cid = lax.axis_index("c")
        sid = lax.axis_index("s")
        w = cid * _NS + sid
        _fill_rows(gbuf.at[0], _CHUNK, d, 0.0)

        @pl.loop(0, _RPT // _CHUNK)
        def _(z):
            pltpu.sync_copy(gbuf.at[0],
                            acc.at[pl.ds(sid * _RPT + z * _CHUNK, _CHUNK)])

        plsc.subcore_barrier()

        def start_gather(j, b):
            pltpu.async_copy(g_hbm.at[src_v.at[j + b]], gbuf.at[b],
                             gsem.at[b])

        def wait_gather(j, b):
            pltpu.make_async_copy(
                g_hbm.at[src_v.at[j + b]], gbuf.at[b], gsem.at[b]).wait()

        def start_scatter(j, b):
            pltpu.async_copy(gbuf.at[b], acc.at[dst_v.at[j + b]], ssem.at[b],
                             add=True)

        def wait_scatter(j, b):
            pltpu.make_async_copy(
                gbuf.at[b], acc.at[dst_v.at[j + b]], ssem.at[b]).wait()

        @pl.loop(0, _CPT // _SEG)
        def _(s):
            base = w * _CPT + s * _SEG
            pltpu.sync_copy(src_hbm.at[pl.ds(base, _SEG)], src_v)
            pltpu.sync_copy(dst_hbm.at[pl.ds(base, _SEG)], dst_v)
            for b in range(_NB):
                start_gather(0, b)

            @pl.loop(0, _SEG - _NB, step=_NB)
            def _(j):
                for b in range(_NB):
                    wait_gather(j, b)
                    start_scatter(j, b)
                for b in range(_NB):
                    wait_scatter(j, b)
                    start_gather(j + _NB, b)

            for b in range(_NB):
                wait_gather(_SEG - _NB, b)
                start_scatter(_SEG - _NB, b)
            for b in range(_NB):
                wait_scatter(_SEG - _NB, b)

        plsc.subcore_barrier()
        pltpu.sync_copy(acc.at[pl.ds(sid * _RPT, _RPT)],
                        out_hbm.at[cid, pl.ds(sid * _RPT, _RPT)])

    return agg_kernel


_sc_deg = _make_sc_deg()
_sc_agg128 = _make_sc_agg(128)


def _deg_specs():
    return [
        pl.BlockSpec((1, _RB, 16), lambda i: (0, i, 0)),
        pl.BlockSpec((1, _RB, 16), lambda i: (1, i, 0)),
    ]


def _dinv(d0_ref, d1_ref):
    return lax.rsqrt(d0_ref[0][:, :1] + d1_ref[0][:, :1] + 1.0)


def _tc_matmul(x, w):
    k = w.shape[1]

    def body(x_ref, w_ref, o_ref):
        o_ref[...] = jnp.dot(x_ref[...], w_ref[...],
                             preferred_element_type=jnp.float32)

    return pl.pallas_call(
        body,
        grid=(_N // _RB,),
        in_specs=[pl.BlockSpec((_RB, 128), lambda i: (i, 0)),
                  pl.BlockSpec((128, k), lambda i: (0, 0))],
        out_specs=pl.BlockSpec((_RB, k), lambda i: (i, 0)),
        out_shape=jax.ShapeDtypeStruct((_N, k), jnp.float32),
    )(x, w)


def _tc_scale(h, deg2):
    def body(h_ref, d0_ref, d1_ref, o_ref):
        o_ref[...] = h_ref[...] * _dinv(d0_ref, d1_ref)

    return pl.pallas_call(
        body,
        grid=(_N // _RB,),
        in_specs=[pl.BlockSpec((_RB, 128), lambda i: (i, 0))] + _deg_specs(),
        out_specs=pl.BlockSpec((_RB, 128), lambda i: (i, 0)),
        out_shape=jax.ShapeDtypeStruct((_N, 128), jnp.float32),
    )(h, deg2, deg2)


def _tc_layer2(agg, g1, deg2, b1, w2):
    def body(a0_ref, a1_ref, g_ref, d0_ref, d1_ref, b_ref, w_ref, o_ref):
        dinv = _dinv(d0_ref, d1_ref)
        g = g_ref[...]
        pre = (a0_ref[0] + a1_ref[0] + g) * dinv + b_ref[...]
        h = jnp.maximum(pre, 0.0)
        g2 = jnp.dot(h, w_ref[...], preferred_element_type=jnp.float32) * dinv
        o_ref[...] = jnp.pad(g2, ((0, 0), (0, 64)))

    return pl.pallas_call(
        body,
        grid=(_N // _RB,),
        in_specs=[pl.BlockSpec((1, _RB, 128), lambda i: (0, i, 0)),
                  pl.BlockSpec((1, _RB, 128), lambda i: (1, i, 0)),
                  pl.BlockSpec((_RB, 128), lambda i: (i, 0))]
                 + _deg_specs()
                 + [pl.BlockSpec((1, 128), lambda i: (0, 0)),
                    pl.BlockSpec((128, 64), lambda i: (0, 0))],
        out_specs=pl.BlockSpec((_RB, 128), lambda i: (i, 0)),
        out_shape=jax.ShapeDtypeStruct((_N, 128), jnp.float32),
    )(agg, agg, g1, deg2, deg2, b1, w2)


def _tc_out(agg, g2, deg2, b2):
    def body(a0_ref, a1_ref, g_ref, d0_ref, d1_ref, b_ref, o_ref):
        dinv = _dinv(d0_ref, d1_ref)
        o = ((a0_ref[0] + a1_ref[0] + g_ref[...]) * dinv)[:, :64] + b_ref[...]
        m = jnp.max(o, axis=1, keepdims=True)
        s = jnp.log(jnp.sum(jnp.exp(o - m), axis=1, keepdims=True))
        o_ref[...] = o - m - s

    return pl.pallas_call(
        body,
        grid=(_N // _RB,),
        in_specs=[pl.BlockSpec((1, _RB, 128), lambda i: (0, i, 0)),
                  pl.BlockSpec((1, _RB, 128), lambda i: (1, i, 0)),
                  pl.BlockSpec((_RB, 128), lambda i: (i, 0))]
                 + _deg_specs()
                 + [pl.BlockSpec((1, 64), lambda i: (0, 0))],
        out_specs=pl.BlockSpec((_RB, 64), lambda i: (i, 0)),
        out_shape=jax.ShapeDtypeStruct((_N, 64), jnp.float32),
    )(agg, agg, g2, deg2, deg2, b2)


def kernel(x, edge_index, W1, b1, W2, b2):
    ei = edge_index.astype(jnp.int32)
    pad = _EPAD - _E
    pad_idx = jnp.arange(pad, dtype=jnp.int32)
    srcp = jnp.concatenate(
        [ei[0], pad_idx % _N]).reshape(_TOT_CHUNKS, _CHUNK)
    dstp = jnp.concatenate(
        [ei[1], _N + pad_idx % (_NACC - _N)]).reshape(_TOT_CHUNKS, _CHUNK)

    deg2 = _sc_deg(dstp)
    h1 = _tc_matmul(x, W1)
    g1 = _tc_scale(h1, deg2)
    agg1 = _sc_agg128(g1, srcp, dstp)
    g2 = _tc_layer2(agg1, g1, deg2, b1.reshape(1, 128), W2)
    agg2 = _sc_agg128(g2, srcp, dstp)
    return _tc_out(agg2, g2, deg2, b2.reshape(1, 64))


# speedup vs baseline: 29.5107x; 1.1546x over previous
import functools

import jax
import jax.numpy as jnp
from jax import lax
from jax.experimental import pallas as pl
from jax.experimental.pallas import tpu as pltpu
from jax.experimental.pallas import tpu_sc as plsc

_N = 10000
_E = 320000
_CHUNK = 64
_NC, _NS = 2, 16
_CPT = 160
_SEG = 40
_TOT_CHUNKS = _NC * _NS * _CPT
_EPAD = _TOT_CHUNKS * _CHUNK
_NACC = 10240
_RPT = _NACC // _NS
_ZR = 16
_NB = 4
_RB = 1000


def _vec_mesh():
    return plsc.VectorSubcoreMesh(
        core_axis_name="c", subcore_axis_name="s",
        num_cores=_NC, num_subcores=_NS)


def _fill_rows(ref, rows, width, value):
    vec = jnp.full((16,), value, jnp.float32)

    @pl.loop(0, rows)
    def _(r):
        row = ref.at[r]
        for c in range(width // 16):
            row[pl.ds(c * 16, 16)] = vec


_DEG_CHUNK = 128
_DEG_CPT = _EPAD // (_NC * _NS * _DEG_CHUNK)


def _make_sc_deg():
    @functools.partial(
        pl.kernel,
        out_type=jax.ShapeDtypeStruct((_NC, _NACC, 16), jnp.float32),
        mesh=_vec_mesh(),
        scratch_types=[
            pltpu.VMEM((_DEG_CPT, _DEG_CHUNK), jnp.int32),
            pltpu.VMEM((_DEG_CHUNK, 16), jnp.float32),
            pltpu.VMEM((_ZR, 16), jnp.float32),
            pltpu.VMEM_SHARED((_NACC, 16), jnp.float32),
            pltpu.SemaphoreType.DMA((_NB,)),
        ])
    def deg_kernel(dst_hbm, out_hbm, dst_v, ones_v, z_v, acc, sem):
        cid = lax.axis_index("c")
        sid = lax.axis_index("s")
        w = cid * _NS + sid
        _fill_rows(ones_v, _DEG_CHUNK, 16, 1.0)
        _fill_rows(z_v, _ZR, 16, 0.0)

        @pl.loop(0, _RPT // _ZR)
        def _(z):
            pltpu.sync_copy(z_v, acc.at[pl.ds(sid * _RPT + z * _ZR, _ZR)])

        pltpu.sync_copy(dst_hbm.at[pl.ds(w * _DEG_CPT, _DEG_CPT)], dst_v)
        plsc.subcore_barrier()

        @pl.loop(0, _DEG_CPT, step=_NB)
        def _(j):
            for b in range(_NB):
                pltpu.async_copy(ones_v, acc.at[dst_v.at[j + b]], sem.at[b],
                                 add=True)
            for b in range(_NB):
                pltpu.make_async_copy(ones_v, acc.at[dst_v.at[j + b]],
                                      sem.at[b]).wait()

        plsc.subcore_barrier()
        pltpu.sync_copy(acc.at[pl.ds(sid * _RPT, _RPT)],
                        out_hbm.at[cid, pl.ds(sid * _RPT, _RPT)])

    return deg_kernel


def _make_sc_agg(d):
    @functools.partial(
        pl.kernel,
        out_type=jax.ShapeDtypeStruct((_NC, _NACC, d), jnp.float32),
        mesh=_vec_mesh(),
        scratch_types=[
            pltpu.VMEM((_SEG, _CHUNK), jnp.int32),
            pltpu.VMEM((_SEG, _CHUNK), jnp.int32),
            pltpu.VMEM((_NB, _CHUNK, d), jnp.float32),
            pltpu.VMEM_SHARED((_NACC, d), jnp.float32),
            pltpu.SemaphoreType.DMA((_NB,)),
            pltpu.SemaphoreType.DMA((_NB,)),
        ])
    def agg_kernel(g_hbm, src_hbm, dst_hbm, out_hbm,
                   src_v, dst_v, gbuf, acc, gsem, ssem):
        cid = lax.axis_index("c")
        sid = lax.axis_index("s")
        w = cid * _NS + sid
        _fill_rows(gbuf.at[0], _CHUNK, d, 0.0)

        @pl.loop(0, _RPT // _CHUNK)
        def _(z):
            pltpu.sync_copy(gbuf.at[0],
                            acc.at[pl.ds(sid * _RPT + z * _CHUNK, _CHUNK)])

        plsc.subcore_barrier()

        def start_gather(j, b):
            pltpu.async_copy(g_hbm.at[src_v.at[j + b]], gbuf.at[b],
                             gsem.at[b])

        def wait_gather(j, b):
            pltpu.make_async_copy(
                g_hbm.at[src_v.at[j + b]], gbuf.at[b], gsem.at[b]).wait()

        def start_scatter(j, b):
            pltpu.async_copy(gbuf.at[b], acc.at[dst_v.at[j + b]], ssem.at[b],
                             add=True)

        def wait_scatter(j, b):
            pltpu.make_async_copy(
                gbuf.at[b], acc.at[dst_v.at[j + b]], ssem.at[b]).wait()

        @pl.loop(0, _CPT // _SEG)
        def _(s):
            base = w * _CPT + s * _SEG
            pltpu.sync_copy(src_hbm.at[pl.ds(base, _SEG)], src_v)
            pltpu.sync_copy(dst_hbm.at[pl.ds(base, _SEG)], dst_v)
            for b in range(_NB):
                start_gather(0, b)

            @pl.loop(0, _SEG - _NB, step=_NB)
            def _(j):
                for b in range(_NB):
                    wait_gather(j, b)
                    start_scatter(j, b)
                for b in range(_NB):
                    wait_scatter(j, b)
                    start_gather(j + _NB, b)

            for b in range(_NB):
                wait_gather(_SEG - _NB, b)
                start_scatter(_SEG - _NB, b)
            for b in range(_NB):
                wait_scatter(_SEG - _NB, b)

        plsc.subcore_barrier()
        pltpu.sync_copy(acc.at[pl.ds(sid * _RPT, _RPT)],
                        out_hbm.at[cid, pl.ds(sid * _RPT, _RPT)])

    return agg_kernel


_sc_deg = _make_sc_deg()
_sc_agg128 = _make_sc_agg(128)


def _deg_specs():
    return [
        pl.BlockSpec((1, _RB, 16), lambda i: (0, i, 0)),
        pl.BlockSpec((1, _RB, 16), lambda i: (1, i, 0)),
    ]


def _dinv(d0_ref, d1_ref):
    return lax.rsqrt(d0_ref[0][:, :1] + d1_ref[0][:, :1] + 1.0)


def _tc_matmul(x, w):
    k = w.shape[1]

    def body(x_ref, w_ref, o_ref):
        o_ref[...] = jnp.dot(x_ref[...], w_ref[...],
                             preferred_element_type=jnp.float32)

    return pl.pallas_call(
        body,
        grid=(_N // _RB,),
        in_specs=[pl.BlockSpec((_RB, 128), lambda i: (i, 0)),
                  pl.BlockSpec((128, k), lambda i: (0, 0))],
        out_specs=pl.BlockSpec((_RB, k), lambda i: (i, 0)),
        out_shape=jax.ShapeDtypeStruct((_N, k), jnp.float32),
    )(x, w)


def _tc_scale(h, deg2):
    def body(h_ref, d0_ref, d1_ref, o_ref):
        o_ref[...] = h_ref[...] * _dinv(d0_ref, d1_ref)

    return pl.pallas_call(
        body,
        grid=(_N // _RB,),
        in_specs=[pl.BlockSpec((_RB, 128), lambda i: (i, 0))] + _deg_specs(),
        out_specs=pl.BlockSpec((_RB, 128), lambda i: (i, 0)),
        out_shape=jax.ShapeDtypeStruct((_N, 128), jnp.float32),
    )(h, deg2, deg2)


def _tc_layer2(agg, g1, deg2, b1, w2):
    def body(a0_ref, a1_ref, g_ref, d0_ref, d1_ref, b_ref, w_ref, o_ref):
        dinv = _dinv(d0_ref, d1_ref)
        g = g_ref[...]
        pre = (a0_ref[0] + a1_ref[0] + g) * dinv + b_ref[...]
        h = jnp.maximum(pre, 0.0)
        g2 = jnp.dot(h, w_ref[...], preferred_element_type=jnp.float32) * dinv
        o_ref[...] = jnp.pad(g2, ((0, 0), (0, 64)))

    return pl.pallas_call(
        body,
        grid=(_N // _RB,),
        in_specs=[pl.BlockSpec((1, _RB, 128), lambda i: (0, i, 0)),
                  pl.BlockSpec((1, _RB, 128), lambda i: (1, i, 0)),
                  pl.BlockSpec((_RB, 128), lambda i: (i, 0))]
                 + _deg_specs()
                 + [pl.BlockSpec((1, 128), lambda i: (0, 0)),
                    pl.BlockSpec((128, 64), lambda i: (0, 0))],
        out_specs=pl.BlockSpec((_RB, 128), lambda i: (i, 0)),
        out_shape=jax.ShapeDtypeStruct((_N, 128), jnp.float32),
    )(agg, agg, g1, deg2, deg2, b1, w2)


def _tc_out(agg, g2, deg2, b2):
    def body(a0_ref, a1_ref, g_ref, d0_ref, d1_ref, b_ref, o_ref):
        dinv = _dinv(d0_ref, d1_ref)
        o = ((a0_ref[0] + a1_ref[0] + g_ref[...]) * dinv)[:, :64] + b_ref[...]
        m = jnp.max(o, axis=1, keepdims=True)
        s = jnp.log(jnp.sum(jnp.exp(o - m), axis=1, keepdims=True))
        o_ref[...] = o - m - s

    return pl.pallas_call(
        body,
        grid=(_N // _RB,),
        in_specs=[pl.BlockSpec((1, _RB, 128), lambda i: (0, i, 0)),
                  pl.BlockSpec((1, _RB, 128), lambda i: (1, i, 0)),
                  pl.BlockSpec((_RB, 128), lambda i: (i, 0))]
                 + _deg_specs()
                 + [pl.BlockSpec((1, 64), lambda i: (0, 0))],
        out_specs=pl.BlockSpec((_RB, 64), lambda i: (i, 0)),
        out_shape=jax.ShapeDtypeStruct((_N, 64), jnp.float32),
    )(agg, agg, g2, deg2, deg2, b2)


def kernel(x, edge_index, W1, b1, W2, b2):
    ei = edge_index.astype(jnp.int32)
    pad = _EPAD - _E
    pad_idx = jnp.arange(pad, dtype=jnp.int32)
    srcp = jnp.concatenate(
        [ei[0], pad_idx % _N]).reshape(_TOT_CHUNKS, _CHUNK)
    dstp = jnp.concatenate(
        [ei[1], _N + pad_idx % (_NACC - _N)]).reshape(_TOT_CHUNKS, _CHUNK)

    deg2 = _sc_deg(dstp.reshape(_EPAD // _DEG_CHUNK, _DEG_CHUNK))
    h1 = _tc_matmul(x, W1)
    g1 = _tc_scale(h1, deg2)
    agg1 = _sc_agg128(g1, srcp, dstp)
    g2 = _tc_layer2(agg1, g1, deg2, b1.reshape(1, 128), W2)
    agg2 = _sc_agg128(g2, srcp, dstp)
    return _tc_out(agg2, g2, deg2, b2.reshape(1, 64))
